# baseline (device time: 31760 ns/iter reference)
import jax
import jax.numpy as jnp
from jax import lax
from jax.experimental import pallas as pl
from jax.experimental.pallas import tpu as pltpu


def kernel(x, router, W1, W2):
    t_loc, d = x.shape
    e_loc, _, f = W1.shape

    def body(x_ref, r_ref, w1_ref, w2_ref, out_ref,
             xfull_ref, rpeer_ref, csend_ref, crecv_ref,
             send_sems, recv_sems):
        my_x = lax.axis_index("x")
        my_y = lax.axis_index("y")
        peer = (1 - my_x, my_y)

        barrier_sem = pltpu.get_barrier_semaphore()
        pl.semaphore_signal(barrier_sem, inc=1, device_id=peer,
                            device_id_type=pl.DeviceIdType.MESH)
        pl.semaphore_wait(barrier_sem, 1)

        rdma_x = pltpu.make_async_remote_copy(
            src_ref=x_ref,
            dst_ref=xfull_ref.at[pl.ds(t_loc, t_loc), :],
            send_sem=send_sems.at[0],
            recv_sem=recv_sems.at[0],
            device_id=peer,
            device_id_type=pl.DeviceIdType.MESH,
        )
        rdma_x.start()
        rdma_r = pltpu.make_async_remote_copy(
            src_ref=r_ref,
            dst_ref=rpeer_ref,
            send_sem=send_sems.at[1],
            recv_sem=recv_sems.at[1],
            device_id=peer,
            device_id_type=pl.DeviceIdType.MESH,
        )
        rdma_r.start()

        xfull_ref[pl.ds(0, t_loc), :] = x_ref[...]

        rdma_x.wait()
        rdma_r.wait()

        X = xfull_ref[...]
        gm = jnp.dot(X, r_ref[...],
                     precision=lax.Precision.HIGHEST,
                     preferred_element_type=jnp.float32)
        gp = jnp.dot(X, rpeer_ref[...],
                     precision=lax.Precision.HIGHEST,
                     preferred_element_type=jnp.float32)
        g = jnp.concatenate([gm, gp], axis=-1)

        m1 = jnp.max(g, axis=-1, keepdims=True)
        g_wo_top1 = jnp.where(g == m1, -1e30, g)
        m2 = jnp.max(g_wo_top1, axis=-1, keepdims=True)
        sel = g >= m2
        ex = jnp.where(sel, jnp.exp(g - m1), 0.0)
        w = ex / jnp.sum(ex, axis=-1, keepdims=True)

        Xb = X.astype(jnp.bfloat16)
        C = jnp.zeros((2 * t_loc, d), jnp.float32)
        for j in range(e_loc):
            H = jnp.maximum(
                jnp.dot(Xb, w1_ref[j].astype(jnp.bfloat16),
                        preferred_element_type=jnp.float32), 0.0)
            P = jnp.dot(H.astype(jnp.bfloat16), w2_ref[j].astype(jnp.bfloat16),
                        preferred_element_type=jnp.float32)
            C = C + w[:, j:j + 1] * P

        csend_ref[...] = C[t_loc:, :]
        rdma_c = pltpu.make_async_remote_copy(
            src_ref=csend_ref,
            dst_ref=crecv_ref,
            send_sem=send_sems.at[2],
            recv_sem=recv_sems.at[2],
            device_id=peer,
            device_id_type=pl.DeviceIdType.MESH,
        )
        rdma_c.start()
        rdma_c.wait()

        out_ref[...] = C[:t_loc, :] + crecv_ref[...]

    return pl.pallas_call(
        body,
        out_shape=jax.ShapeDtypeStruct((t_loc, d), jnp.float32),
        in_specs=[pl.BlockSpec(memory_space=pltpu.VMEM)] * 4,
        out_specs=pl.BlockSpec(memory_space=pltpu.VMEM),
        scratch_shapes=[
            pltpu.VMEM((2 * t_loc, d), jnp.float32),
            pltpu.VMEM((d, e_loc), jnp.float32),
            pltpu.VMEM((t_loc, d), jnp.float32),
            pltpu.VMEM((t_loc, d), jnp.float32),
            pltpu.SemaphoreType.DMA((3,)),
            pltpu.SemaphoreType.DMA((3,)),
        ],
        compiler_params=pltpu.CompilerParams(collective_id=0),
    )(x, router, W1, W2)


# device time: 26968 ns/iter; 1.1777x vs baseline; 1.1777x over previous
import jax
import jax.numpy as jnp
from jax import lax
from jax.experimental import pallas as pl
from jax.experimental.pallas import tpu as pltpu

N_CHUNK = 2


def kernel(x, router, W1, W2):
    t_loc, d = x.shape
    e_loc, _, f = W1.shape
    t_chk = t_loc // N_CHUNK

    def body(x_ref, r_ref, w1_ref, w2_ref, out_ref,
             xfull_ref, rpeer_ref, wsend_ref, wrecv_ref,
             csend_ref, crecv_ref, send_sems, recv_sems):
        my_x = lax.axis_index("x")
        my_y = lax.axis_index("y")
        peer = (1 - my_x, my_y)

        def exchange(src, dst, sem):
            return pltpu.make_async_remote_copy(
                src_ref=src, dst_ref=dst,
                send_sem=send_sems.at[sem], recv_sem=recv_sems.at[sem],
                device_id=peer, device_id_type=pl.DeviceIdType.MESH,
            )

        barrier_sem = pltpu.get_barrier_semaphore()
        pl.semaphore_signal(barrier_sem, inc=1, device_id=peer,
                            device_id_type=pl.DeviceIdType.MESH)
        pl.semaphore_wait(barrier_sem, 1)

        rdma_r = exchange(r_ref, rpeer_ref, 0)
        rdma_r.start()

        xmb = x_ref[...].astype(jnp.bfloat16)
        xfull_ref[pl.ds(0, t_loc), :] = xmb
        gmm = jnp.dot(x_ref[...], r_ref[...],
                      precision=lax.Precision.HIGHEST,
                      preferred_element_type=jnp.float32)

        rdma_r.wait()

        gmp = jnp.dot(x_ref[...], rpeer_ref[...],
                      precision=lax.Precision.HIGHEST,
                      preferred_element_type=jnp.float32)
        g = jnp.concatenate([gmm, gmp], axis=-1)
        m1 = jnp.max(g, axis=-1, keepdims=True)
        m2 = jnp.max(jnp.where(g == m1, -1e30, g), axis=-1, keepdims=True)
        sel = g >= m2
        ex = jnp.where(sel, jnp.exp(g - m1), 0.0)
        w_mine = ex / jnp.sum(ex, axis=-1, keepdims=True)

        wsend_ref[...] = w_mine[:, e_loc:]
        rdma_x = exchange(xfull_ref.at[pl.ds(0, t_loc), :],
                          xfull_ref.at[pl.ds(t_loc, t_loc), :], 1)
        rdma_x.start()
        rdma_w = exchange(wsend_ref, wrecv_ref, 2)
        rdma_w.start()

        w1b = [w1_ref[j].astype(jnp.bfloat16) for j in range(e_loc)]
        w2b = [w2_ref[j].astype(jnp.bfloat16) for j in range(e_loc)]

        def experts(xb, wcols):
            acc = jnp.zeros((xb.shape[0], d), jnp.float32)
            for j in range(e_loc):
                h = jnp.maximum(
                    jnp.dot(xb, w1b[j], preferred_element_type=jnp.float32),
                    0.0)
                p = jnp.dot(h.astype(jnp.bfloat16), w2b[j],
                            preferred_element_type=jnp.float32)
                acc = acc + wcols[:, j:j + 1] * p
            return acc

        c_mine = experts(xmb, w_mine)

        rdma_x.wait()
        rdma_w.wait()

        xpb = xfull_ref[pl.ds(t_loc, t_loc), :]
        wp = wrecv_ref[...]
        rdma_c = []
        for c in range(N_CHUNK):
            rows = slice(c * t_chk, (c + 1) * t_chk)
            cc = experts(xpb[rows, :], wp[rows, :])
            csend_ref[c, :, :] = cc.astype(jnp.bfloat16)
            rc = exchange(csend_ref.at[c], crecv_ref.at[c], 3 + c)
            rc.start()
            rdma_c.append(rc)
        for rc in rdma_c:
            rc.wait()

        crecv = crecv_ref[...].reshape(t_loc, d).astype(jnp.float32)
        out_ref[...] = c_mine + crecv

    return pl.pallas_call(
        body,
        out_shape=jax.ShapeDtypeStruct((t_loc, d), jnp.float32),
        in_specs=[pl.BlockSpec(memory_space=pltpu.VMEM)] * 4,
        out_specs=pl.BlockSpec(memory_space=pltpu.VMEM),
        scratch_shapes=[
            pltpu.VMEM((2 * t_loc, d), jnp.bfloat16),
            pltpu.VMEM((d, e_loc), jnp.float32),
            pltpu.VMEM((t_loc, e_loc), jnp.float32),
            pltpu.VMEM((t_loc, e_loc), jnp.float32),
            pltpu.VMEM((N_CHUNK, t_chk, d), jnp.bfloat16),
            pltpu.VMEM((N_CHUNK, t_chk, d), jnp.bfloat16),
            pltpu.SemaphoreType.DMA((3 + N_CHUNK,)),
            pltpu.SemaphoreType.DMA((3 + N_CHUNK,)),
        ],
        compiler_params=pltpu.CompilerParams(collective_id=0),
    )(x, router, W1, W2)
